# baseline (speedup 1.0000x reference)
import functools

import jax
import jax.numpy as jnp
from jax import lax
from jax.experimental import pallas as pl
from jax.experimental.pallas import tpu as pltpu
from jax.experimental.pallas import tpu_sc as plsc

N = 10000
E = 320000
D = 128

NC = 2
NS = 16
NW = NC * NS
EPW = E // NW
K = 80
CH = EPW // K
RPT = 624
TAIL = N - NS * RPT
ZR = 24


def _scatter_body(h_hbm, pk_hbm, out_hbm,
                  pk, sb0, db0, sb1, db1, rows0, rows1, zbuf, acc,
                  sem0, sem1, semP, semZ):
    c = lax.axis_index("c")
    s = lax.axis_index("s")
    wid = c * NS + s

    pltpu.async_copy(pk_hbm.at[wid], pk, semP)

    for r in range(ZR):
        for q in range(D // 16):
            zbuf[r, pl.ds(q * 16, 16)] = jnp.zeros((16,), jnp.float32)
    row0 = s * RPT
    for t in range(RPT // ZR):
        pltpu.async_copy(zbuf, acc.at[pl.ds(row0 + t * ZR, ZR)], semZ)

    @pl.when(s == 0)
    def _zero_tail():
        pltpu.async_copy(zbuf.at[pl.ds(0, TAIL)], acc.at[pl.ds(NS * RPT, TAIL)],
                         semZ)

    pltpu.make_async_copy(pk_hbm.at[wid], pk, semP).wait()

    def unpack(k, sb, db):
        for q in range(K // 16):
            v = pk[k, pl.ds(q * 16, 16)]
            sb[pl.ds(q * 16, 16)] = v >> 14
            db[pl.ds(q * 16, 16)] = v & 16383

    def gather(sb, buf, sem):
        pltpu.async_copy(h_hbm.at[sb], buf, sem)

    def gwait(sb, buf, sem):
        pltpu.make_async_copy(h_hbm.at[sb], buf, sem).wait()

    def scat(buf, db):
        pltpu.sync_copy(buf, acc.at[db], add=True)

    unpack(0, sb0, db0)
    gather(sb0, rows0, sem0)

    for t in range(RPT // ZR):
        pltpu.make_async_copy(zbuf, acc.at[pl.ds(row0 + t * ZR, ZR)],
                              semZ).wait()

    @pl.when(s == 0)
    def _zero_tail_wait():
        pltpu.make_async_copy(zbuf.at[pl.ds(0, TAIL)],
                              acc.at[pl.ds(NS * RPT, TAIL)], semZ).wait()

    plsc.subcore_barrier()

    def pipe(j, carry):
        k0 = 2 * j
        unpack(k0 + 1, sb1, db1)
        gather(sb1, rows1, sem1)
        gwait(sb0, rows0, sem0)
        scat(rows0, db0)
        unpack(k0 + 2, sb0, db0)
        gather(sb0, rows0, sem0)
        gwait(sb1, rows1, sem1)
        scat(rows1, db1)
        return carry

    lax.fori_loop(0, (CH - 1) // 2, pipe, 0)
    gwait(sb0, rows0, sem0)
    scat(rows0, db0)
    plsc.subcore_barrier()

    pltpu.sync_copy(acc.at[pl.ds(row0, RPT)],
                    out_hbm.at[c, pl.ds(row0, RPT)])

    @pl.when(s == 0)
    def _write_tail():
        pltpu.sync_copy(acc.at[pl.ds(NS * RPT, TAIL)],
                        out_hbm.at[c, pl.ds(NS * RPT, TAIL)])


@jax.jit
def _scatter_partials(h, packed):
    mesh = plsc.VectorSubcoreMesh(core_axis_name="c", subcore_axis_name="s")
    f = pl.kernel(
        _scatter_body,
        out_type=jax.ShapeDtypeStruct((NC, N, D), jnp.float32),
        mesh=mesh,
        scratch_types=[
            pltpu.VMEM((CH, K), jnp.int32),
            pltpu.VMEM((K,), jnp.int32),
            pltpu.VMEM((K,), jnp.int32),
            pltpu.VMEM((K,), jnp.int32),
            pltpu.VMEM((K,), jnp.int32),
            pltpu.VMEM((K, D), jnp.float32),
            pltpu.VMEM((K, D), jnp.float32),
            pltpu.VMEM((ZR, D), jnp.float32),
            pltpu.VMEM_SHARED((N, D), jnp.float32),
            pltpu.SemaphoreType.DMA,
            pltpu.SemaphoreType.DMA,
            pltpu.SemaphoreType.DMA,
            pltpu.SemaphoreType.DMA,
        ],
    )
    return f(h, packed)


BN = 1000


def _mm_pre_body(x_ref, w_ref, b_ref, e_ref, o_ref, pk_ref):
    acc = lax.dot_general(x_ref[...], w_ref[...],
                          dimension_numbers=(((1,), (1,)), ((), ())),
                          preferred_element_type=jnp.float32,
                          precision=lax.Precision.HIGHEST)
    o_ref[...] = acc + b_ref[...]
    pk_ref[...] = (e_ref[0:1, :] << 14) | e_ref[1:2, :]


def _mm_agg_body(x_ref, p0_ref, p1_ref, w_ref, b_ref, o_ref, *, relu):
    hh = x_ref[...] + p0_ref[0] + p1_ref[0]
    acc = lax.dot_general(hh, w_ref[...],
                          dimension_numbers=(((1,), (1,)), ((), ())),
                          preferred_element_type=jnp.float32,
                          precision=lax.Precision.HIGHEST)
    acc = acc + b_ref[...]
    o_ref[...] = jnp.maximum(acc, 0.0) if relu else acc


_row_spec = pl.BlockSpec((BN, D), lambda i: (i, 0))
_p0_spec = pl.BlockSpec((1, BN, D), lambda i: (0, i, 0))
_p1_spec = pl.BlockSpec((1, BN, D), lambda i: (1, i, 0))
_full_spec = pl.BlockSpec((D, D), lambda i: (0, 0))
_b_spec = pl.BlockSpec((1, D), lambda i: (0, 0))


EB = E // (N // BN)


def _linear_pre(x, w, b, edge_index):
    return pl.pallas_call(
        _mm_pre_body,
        grid=(N // BN,),
        in_specs=[_row_spec, _full_spec, _b_spec,
                  pl.BlockSpec((2, EB), lambda i: (0, i))],
        out_specs=[_row_spec, pl.BlockSpec((1, EB), lambda i: (0, i))],
        out_shape=[jax.ShapeDtypeStruct((N, D), jnp.float32),
                   jax.ShapeDtypeStruct((1, E), jnp.int32)],
    )(x, w, b.reshape(1, D), edge_index)


def _linear_agg(x, p, w, b, relu=False):
    return pl.pallas_call(
        functools.partial(_mm_agg_body, relu=relu),
        grid=(N // BN,),
        in_specs=[_row_spec, _p0_spec, _p1_spec, _full_spec, _b_spec],
        out_specs=_row_spec,
        out_shape=jax.ShapeDtypeStruct((N, D), jnp.float32),
    )(x, p, p, w, b.reshape(1, D))


def kernel(x, edge_index, W_pre, b_pre, W1, b1, W2, b2):
    h0, packed = _linear_pre(x, W_pre, b_pre, edge_index)
    packed = packed.reshape(NW, CH, K)
    p = _scatter_partials(h0, packed)
    h1 = _linear_agg(h0, p, W1, b1, relu=True)
    q = _scatter_partials(h1, packed)
    return _linear_agg(h1, q, W2, b2, relu=False)


# speedup vs baseline: 1.0127x; 1.0127x over previous
import functools

import jax
import jax.numpy as jnp
from jax import lax
from jax.experimental import pallas as pl
from jax.experimental.pallas import tpu as pltpu
from jax.experimental.pallas import tpu_sc as plsc

N = 10000
E = 320000
D = 128

NC = 2
NS = 16
NW = NC * NS
EPW = E // NW
K = 80
CH = EPW // K
RPT = 624
TAIL = N - NS * RPT
ZR = 24


def _scatter_body(h_hbm, pk_hbm, out_hbm,
                  pk, sb0, db0, sb1, db1, rows0, rows1, zbuf, acc,
                  sem0, sem1, semP, semZ):
    c = lax.axis_index("c")
    s = lax.axis_index("s")
    wid = c * NS + s

    pltpu.async_copy(pk_hbm.at[wid], pk, semP)

    for r in range(ZR):
        for q in range(D // 16):
            zbuf[r, pl.ds(q * 16, 16)] = jnp.zeros((16,), jnp.float32)
    row0 = s * RPT
    for t in range(RPT // ZR):
        pltpu.async_copy(zbuf, acc.at[pl.ds(row0 + t * ZR, ZR)], semZ)

    @pl.when(s == 0)
    def _zero_tail():
        pltpu.async_copy(zbuf.at[pl.ds(0, TAIL)], acc.at[pl.ds(NS * RPT, TAIL)],
                         semZ)

    pltpu.make_async_copy(pk_hbm.at[wid], pk, semP).wait()

    def unpack(k, sb, db):
        for q in range(K // 16):
            v = pk[k, pl.ds(q * 16, 16)]
            sb[pl.ds(q * 16, 16)] = v >> 14
            db[pl.ds(q * 16, 16)] = v & 16383

    def gather(sb, buf, sem):
        pltpu.async_copy(h_hbm.at[sb], buf, sem)

    def gwait(sb, buf, sem):
        pltpu.make_async_copy(h_hbm.at[sb], buf, sem).wait()

    def scat(buf, db):
        pltpu.sync_copy(buf, acc.at[db], add=True)

    unpack(0, sb0, db0)
    gather(sb0, rows0, sem0)

    for t in range(RPT // ZR):
        pltpu.make_async_copy(zbuf, acc.at[pl.ds(row0 + t * ZR, ZR)],
                              semZ).wait()

    @pl.when(s == 0)
    def _zero_tail_wait():
        pltpu.make_async_copy(zbuf.at[pl.ds(0, TAIL)],
                              acc.at[pl.ds(NS * RPT, TAIL)], semZ).wait()

    plsc.subcore_barrier()

    def pipe(j, carry):
        k0 = 2 * j
        unpack(k0 + 1, sb1, db1)
        gather(sb1, rows1, sem1)
        gwait(sb0, rows0, sem0)
        scat(rows0, db0)
        unpack(k0 + 2, sb0, db0)
        gather(sb0, rows0, sem0)
        gwait(sb1, rows1, sem1)
        scat(rows1, db1)
        return carry

    lax.fori_loop(0, (CH - 1) // 2, pipe, 0)
    gwait(sb0, rows0, sem0)
    scat(rows0, db0)
    plsc.subcore_barrier()

    pltpu.sync_copy(acc.at[pl.ds(row0, RPT)],
                    out_hbm.at[c, pl.ds(row0, RPT)])

    @pl.when(s == 0)
    def _write_tail():
        pltpu.sync_copy(acc.at[pl.ds(NS * RPT, TAIL)],
                        out_hbm.at[c, pl.ds(NS * RPT, TAIL)])


@jax.jit
def _scatter_partials(h, packed):
    mesh = plsc.VectorSubcoreMesh(core_axis_name="c", subcore_axis_name="s")
    f = pl.kernel(
        _scatter_body,
        out_type=jax.ShapeDtypeStruct((NC, N, D), jnp.float32),
        mesh=mesh,
        scratch_types=[
            pltpu.VMEM((CH, K), jnp.int32),
            pltpu.VMEM((K,), jnp.int32),
            pltpu.VMEM((K,), jnp.int32),
            pltpu.VMEM((K,), jnp.int32),
            pltpu.VMEM((K,), jnp.int32),
            pltpu.VMEM((K, D), jnp.float32),
            pltpu.VMEM((K, D), jnp.float32),
            pltpu.VMEM((ZR, D), jnp.float32),
            pltpu.VMEM_SHARED((N, D), jnp.float32),
            pltpu.SemaphoreType.DMA,
            pltpu.SemaphoreType.DMA,
            pltpu.SemaphoreType.DMA,
            pltpu.SemaphoreType.DMA,
        ],
    )
    return f(h, packed)


BN = 5000


def _mm_pre_body(x_ref, w_ref, b_ref, e_ref, o_ref, pk_ref):
    acc = lax.dot_general(x_ref[...], w_ref[...],
                          dimension_numbers=(((1,), (1,)), ((), ())),
                          preferred_element_type=jnp.float32,
                          precision=lax.Precision.HIGHEST)
    o_ref[...] = acc + b_ref[...]
    pk_ref[...] = (e_ref[0:1, :] << 14) | e_ref[1:2, :]


def _mm_agg_body(x_ref, p0_ref, p1_ref, w_ref, b_ref, o_ref, *, relu):
    hh = x_ref[...] + p0_ref[0] + p1_ref[0]
    acc = lax.dot_general(hh, w_ref[...],
                          dimension_numbers=(((1,), (1,)), ((), ())),
                          preferred_element_type=jnp.float32,
                          precision=lax.Precision.HIGHEST)
    acc = acc + b_ref[...]
    o_ref[...] = jnp.maximum(acc, 0.0) if relu else acc


_row_spec = pl.BlockSpec((BN, D), lambda i: (i, 0))
_p0_spec = pl.BlockSpec((1, BN, D), lambda i: (0, i, 0))
_p1_spec = pl.BlockSpec((1, BN, D), lambda i: (1, i, 0))
_full_spec = pl.BlockSpec((D, D), lambda i: (0, 0))
_b_spec = pl.BlockSpec((1, D), lambda i: (0, 0))


EB = E // (N // BN)


def _linear_pre(x, w, b, edge_index):
    return pl.pallas_call(
        _mm_pre_body,
        grid=(N // BN,),
        in_specs=[_row_spec, _full_spec, _b_spec,
                  pl.BlockSpec((2, EB), lambda i: (0, i))],
        out_specs=[_row_spec, pl.BlockSpec((1, EB), lambda i: (0, i))],
        out_shape=[jax.ShapeDtypeStruct((N, D), jnp.float32),
                   jax.ShapeDtypeStruct((1, E), jnp.int32)],
    )(x, w, b.reshape(1, D), edge_index)


def _linear_agg(x, p, w, b, relu=False):
    return pl.pallas_call(
        functools.partial(_mm_agg_body, relu=relu),
        grid=(N // BN,),
        in_specs=[_row_spec, _p0_spec, _p1_spec, _full_spec, _b_spec],
        out_specs=_row_spec,
        out_shape=jax.ShapeDtypeStruct((N, D), jnp.float32),
    )(x, p, p, w, b.reshape(1, D))


def kernel(x, edge_index, W_pre, b_pre, W1, b1, W2, b2):
    h0, packed = _linear_pre(x, W_pre, b_pre, edge_index)
    packed = packed.reshape(NW, CH, K)
    p = _scatter_partials(h0, packed)
    h1 = _linear_agg(h0, p, W1, b1, relu=True)
    q = _scatter_partials(h1, packed)
    return _linear_agg(h1, q, W2, b2, relu=False)


# speedup vs baseline: 1.0428x; 1.0298x over previous
import functools

import jax
import jax.numpy as jnp
from jax import lax
from jax.experimental import pallas as pl
from jax.experimental.pallas import tpu as pltpu
from jax.experimental.pallas import tpu_sc as plsc

N = 10000
E = 320000
D = 128

NC = 2
NS = 16
NW = NC * NS
EPW = E // NW
K = 80
CH = EPW // K
RPT = 624
TAIL = N - NS * RPT
ZR = 24


def _scatter_body(h_hbm, pk_hbm, out_hbm,
                  pk, sb0, db0, sb1, db1, rows0, rows1, zbuf, acc,
                  sem0, sem1, semP, semZ):
    c = lax.axis_index("c")
    s = lax.axis_index("s")
    wid = c * NS + s

    pltpu.async_copy(pk_hbm.at[wid], pk, semP)

    for r in range(ZR):
        for q in range(D // 16):
            zbuf[r, pl.ds(q * 16, 16)] = jnp.zeros((16,), jnp.float32)
    row0 = s * RPT
    for t in range(RPT // ZR):
        pltpu.async_copy(zbuf, acc.at[pl.ds(row0 + t * ZR, ZR)], semZ)

    @pl.when(s == 0)
    def _zero_tail():
        pltpu.async_copy(zbuf.at[pl.ds(0, TAIL)], acc.at[pl.ds(NS * RPT, TAIL)],
                         semZ)

    pltpu.make_async_copy(pk_hbm.at[wid], pk, semP).wait()

    def unpack(k, sb, db):
        for q in range(K // 16):
            v = pk[k, pl.ds(q * 16, 16)]
            sb[pl.ds(q * 16, 16)] = v >> 14
            db[pl.ds(q * 16, 16)] = v & 16383

    def gather(sb, buf, sem):
        pltpu.async_copy(h_hbm.at[sb], buf, sem)

    def gwait(sb, buf, sem):
        pltpu.make_async_copy(h_hbm.at[sb], buf, sem).wait()

    def scat(buf, db):
        pltpu.sync_copy(buf, acc.at[db], add=True)

    unpack(0, sb0, db0)
    gather(sb0, rows0, sem0)

    for t in range(RPT // ZR):
        pltpu.make_async_copy(zbuf, acc.at[pl.ds(row0 + t * ZR, ZR)],
                              semZ).wait()

    @pl.when(s == 0)
    def _zero_tail_wait():
        pltpu.make_async_copy(zbuf.at[pl.ds(0, TAIL)],
                              acc.at[pl.ds(NS * RPT, TAIL)], semZ).wait()

    plsc.subcore_barrier()

    def pipe(j, carry):
        k0 = 2 * j
        unpack(k0 + 1, sb1, db1)
        gather(sb1, rows1, sem1)
        gwait(sb0, rows0, sem0)
        scat(rows0, db0)
        unpack(k0 + 2, sb0, db0)
        gather(sb0, rows0, sem0)
        gwait(sb1, rows1, sem1)
        scat(rows1, db1)
        return carry

    lax.fori_loop(0, (CH - 1) // 2, pipe, 0)
    gwait(sb0, rows0, sem0)
    scat(rows0, db0)
    plsc.subcore_barrier()

    pltpu.sync_copy(acc.at[pl.ds(row0, RPT)],
                    out_hbm.at[c, pl.ds(row0, RPT)])

    @pl.when(s == 0)
    def _write_tail():
        pltpu.sync_copy(acc.at[pl.ds(NS * RPT, TAIL)],
                        out_hbm.at[c, pl.ds(NS * RPT, TAIL)])


@jax.jit
def _scatter_partials(h, packed):
    mesh = plsc.VectorSubcoreMesh(core_axis_name="c", subcore_axis_name="s")
    f = pl.kernel(
        _scatter_body,
        out_type=jax.ShapeDtypeStruct((NC, N, D), jnp.float32),
        mesh=mesh,
        scratch_types=[
            pltpu.VMEM((CH, K), jnp.int32),
            pltpu.VMEM((K,), jnp.int32),
            pltpu.VMEM((K,), jnp.int32),
            pltpu.VMEM((K,), jnp.int32),
            pltpu.VMEM((K,), jnp.int32),
            pltpu.VMEM((K, D), jnp.float32),
            pltpu.VMEM((K, D), jnp.float32),
            pltpu.VMEM((ZR, D), jnp.float32),
            pltpu.VMEM_SHARED((N, D), jnp.float32),
            pltpu.SemaphoreType.DMA,
            pltpu.SemaphoreType.DMA,
            pltpu.SemaphoreType.DMA,
            pltpu.SemaphoreType.DMA,
        ],
    )
    return f(h, packed)


BN = 2000


def _mm_pre_body(x_ref, w_ref, b_ref, e_ref, o_ref, pk_ref):
    acc = lax.dot_general(x_ref[...], w_ref[...],
                          dimension_numbers=(((1,), (1,)), ((), ())),
                          preferred_element_type=jnp.float32,
                          precision=lax.Precision.HIGHEST)
    o_ref[...] = acc + b_ref[...]
    pk_ref[...] = (e_ref[0:1, :] << 14) | e_ref[1:2, :]


def _mm_agg_body(x_ref, p0_ref, p1_ref, w_ref, b_ref, o_ref, *, relu):
    hh = x_ref[...] + p0_ref[0] + p1_ref[0]
    acc = lax.dot_general(hh, w_ref[...],
                          dimension_numbers=(((1,), (1,)), ((), ())),
                          preferred_element_type=jnp.float32,
                          precision=lax.Precision.HIGHEST)
    acc = acc + b_ref[...]
    o_ref[...] = jnp.maximum(acc, 0.0) if relu else acc


_row_spec = pl.BlockSpec((BN, D), lambda i: (i, 0))
_p0_spec = pl.BlockSpec((1, BN, D), lambda i: (0, i, 0))
_p1_spec = pl.BlockSpec((1, BN, D), lambda i: (1, i, 0))
_full_spec = pl.BlockSpec((D, D), lambda i: (0, 0))
_b_spec = pl.BlockSpec((1, D), lambda i: (0, 0))


EB = E // (N // BN)


def _linear_pre(x, w, b, edge_index):
    return pl.pallas_call(
        _mm_pre_body,
        grid=(N // BN,),
        in_specs=[_row_spec, _full_spec, _b_spec,
                  pl.BlockSpec((2, EB), lambda i: (0, i))],
        out_specs=[_row_spec, pl.BlockSpec((1, EB), lambda i: (0, i))],
        out_shape=[jax.ShapeDtypeStruct((N, D), jnp.float32),
                   jax.ShapeDtypeStruct((1, E), jnp.int32)],
    )(x, w, b.reshape(1, D), edge_index)


def _linear_agg(x, p, w, b, relu=False):
    return pl.pallas_call(
        functools.partial(_mm_agg_body, relu=relu),
        grid=(N // BN,),
        in_specs=[_row_spec, _p0_spec, _p1_spec, _full_spec, _b_spec],
        out_specs=_row_spec,
        out_shape=jax.ShapeDtypeStruct((N, D), jnp.float32),
    )(x, p, p, w, b.reshape(1, D))


def kernel(x, edge_index, W_pre, b_pre, W1, b1, W2, b2):
    h0, packed = _linear_pre(x, W_pre, b_pre, edge_index)
    packed = packed.reshape(NW, CH, K)
    p = _scatter_partials(h0, packed)
    h1 = _linear_agg(h0, p, W1, b1, relu=True)
    q = _scatter_partials(h1, packed)
    return _linear_agg(h1, q, W2, b2, relu=False)


# speedup vs baseline: 1.1269x; 1.0806x over previous
import functools

import jax
import jax.numpy as jnp
from jax import lax
from jax.experimental import pallas as pl
from jax.experimental.pallas import tpu as pltpu
from jax.experimental.pallas import tpu_sc as plsc

N = 10000
E = 320000
D = 128

NC = 2
NS = 16
NW = NC * NS
EPW = E // NW
K = 128
CH = EPW // K
KT = EPW - CH * K
RPT = 624
TAIL = N - NS * RPT
ZR = 24


def _scatter_body(h_hbm, pk_hbm, out_hbm,
                  pkb, sbb, dbb, rows, pkT, sbT, dbT, rowsT, zbuf, acc,
                  semI, semG):
    c = lax.axis_index("c")
    s = lax.axis_index("s")
    wid = c * NS + s
    base = wid * EPW

    def pkload(k, b):
        pltpu.async_copy(pk_hbm.at[pl.ds(base + k * K, K)], pkb[b], semI[b])

    def iwait(k, b):
        pltpu.make_async_copy(pk_hbm.at[pl.ds(base + k * K, K)], pkb[b],
                              semI[b]).wait()

    def unpack(b):
        for q in range(K // 16):
            v = pkb[b][pl.ds(q * 16, 16)]
            sbb[b][pl.ds(q * 16, 16)] = v >> 14
            dbb[b][pl.ds(q * 16, 16)] = v & 16383

    def gather(b):
        pltpu.async_copy(h_hbm.at[sbb[b]], rows[b], semG[b])

    def gwait(b):
        pltpu.make_async_copy(h_hbm.at[sbb[b]], rows[b], semG[b]).wait()

    def scat(b):
        pltpu.sync_copy(rows[b], acc.at[dbb[b]], add=True)

    pkload(0, 0)
    pkload(1, 1)

    for r in range(ZR):
        for q in range(D // 16):
            zbuf[r, pl.ds(q * 16, 16)] = jnp.zeros((16,), jnp.float32)
    row0 = s * RPT
    for t in range(RPT // ZR):
        pltpu.async_copy(zbuf, acc.at[pl.ds(row0 + t * ZR, ZR)], semG[0])

    @pl.when(s == 0)
    def _zero_tail():
        pltpu.async_copy(zbuf.at[pl.ds(0, TAIL)], acc.at[pl.ds(NS * RPT, TAIL)],
                         semG[0])

    for t in range(RPT // ZR):
        pltpu.make_async_copy(zbuf, acc.at[pl.ds(row0 + t * ZR, ZR)],
                              semG[0]).wait()

    @pl.when(s == 0)
    def _zero_tail_wait():
        pltpu.make_async_copy(zbuf.at[pl.ds(0, TAIL)],
                              acc.at[pl.ds(NS * RPT, TAIL)], semG[0]).wait()

    iwait(0, 0)
    unpack(0)
    gather(0)
    pkload(2, 0)
    plsc.subcore_barrier()

    def pipe(j, carry):
        k0 = 2 * j
        iwait(k0 + 1, 1)
        unpack(1)
        gather(1)

        @pl.when(k0 + 3 < CH)
        def _pf1():
            pkload(k0 + 3, 1)

        gwait(0)
        scat(0)
        iwait(k0 + 2, 0)
        unpack(0)
        gather(0)

        @pl.when(k0 + 4 < CH)
        def _pf0():
            pkload(k0 + 4, 0)

        gwait(1)
        scat(1)
        return carry

    lax.fori_loop(0, CH // 2 - 1, pipe, 0)

    iwait(CH - 1, 1)
    unpack(1)
    gather(1)
    gwait(0)
    scat(0)
    gwait(1)
    scat(1)

    pltpu.async_copy(pk_hbm.at[pl.ds(base + CH * K, KT)], pkT, semI[0])
    pltpu.make_async_copy(pk_hbm.at[pl.ds(base + CH * K, KT)], pkT,
                          semI[0]).wait()
    vT = pkT[pl.ds(0, 16)]
    sbT[pl.ds(0, 16)] = vT >> 14
    dbT[pl.ds(0, 16)] = vT & 16383
    pltpu.async_copy(h_hbm.at[sbT], rowsT, semG[0])
    pltpu.make_async_copy(h_hbm.at[sbT], rowsT, semG[0]).wait()
    pltpu.sync_copy(rowsT, acc.at[dbT], add=True)

    plsc.subcore_barrier()

    pltpu.sync_copy(acc.at[pl.ds(row0, RPT)],
                    out_hbm.at[c, pl.ds(row0, RPT)])

    @pl.when(s == 0)
    def _write_tail():
        pltpu.sync_copy(acc.at[pl.ds(NS * RPT, TAIL)],
                        out_hbm.at[c, pl.ds(NS * RPT, TAIL)])


@jax.jit
def _scatter_partials(h, packed):
    mesh = plsc.VectorSubcoreMesh(core_axis_name="c", subcore_axis_name="s")
    f = pl.kernel(
        _scatter_body,
        out_type=jax.ShapeDtypeStruct((NC, N, D), jnp.float32),
        mesh=mesh,
        scratch_types=[
            [pltpu.VMEM((K,), jnp.int32) for _ in range(2)],
            [pltpu.VMEM((K,), jnp.int32) for _ in range(2)],
            [pltpu.VMEM((K,), jnp.int32) for _ in range(2)],
            [pltpu.VMEM((K, D), jnp.float32) for _ in range(2)],
            pltpu.VMEM((KT,), jnp.int32),
            pltpu.VMEM((KT,), jnp.int32),
            pltpu.VMEM((KT,), jnp.int32),
            pltpu.VMEM((KT, D), jnp.float32),
            pltpu.VMEM((ZR, D), jnp.float32),
            pltpu.VMEM_SHARED((N, D), jnp.float32),
            [pltpu.SemaphoreType.DMA for _ in range(2)],
            [pltpu.SemaphoreType.DMA for _ in range(2)],
        ],
    )
    return f(h, packed)


BN = 2000


def _mm_pre_body(x_ref, w_ref, b_ref, e_ref, o_ref, pk_ref):
    acc = lax.dot_general(x_ref[...], w_ref[...],
                          dimension_numbers=(((1,), (1,)), ((), ())),
                          preferred_element_type=jnp.float32,
                          precision=lax.Precision.HIGHEST)
    o_ref[...] = acc + b_ref[...]
    pk_ref[...] = (e_ref[0:1, :] << 14) | e_ref[1:2, :]


def _mm_agg_body(x_ref, p0_ref, p1_ref, w_ref, b_ref, o_ref, *, relu):
    hh = x_ref[...] + p0_ref[0] + p1_ref[0]
    acc = lax.dot_general(hh, w_ref[...],
                          dimension_numbers=(((1,), (1,)), ((), ())),
                          preferred_element_type=jnp.float32,
                          precision=lax.Precision.HIGHEST)
    acc = acc + b_ref[...]
    o_ref[...] = jnp.maximum(acc, 0.0) if relu else acc


_row_spec = pl.BlockSpec((BN, D), lambda i: (i, 0))
_p0_spec = pl.BlockSpec((1, BN, D), lambda i: (0, i, 0))
_p1_spec = pl.BlockSpec((1, BN, D), lambda i: (1, i, 0))
_full_spec = pl.BlockSpec((D, D), lambda i: (0, 0))
_b_spec = pl.BlockSpec((1, D), lambda i: (0, 0))


EB = E // (N // BN)


def _linear_pre(x, w, b, edge_index):
    return pl.pallas_call(
        _mm_pre_body,
        grid=(N // BN,),
        in_specs=[_row_spec, _full_spec, _b_spec,
                  pl.BlockSpec((2, EB), lambda i: (0, i))],
        out_specs=[_row_spec, pl.BlockSpec((1, EB), lambda i: (0, i))],
        out_shape=[jax.ShapeDtypeStruct((N, D), jnp.float32),
                   jax.ShapeDtypeStruct((1, E), jnp.int32)],
    )(x, w, b.reshape(1, D), edge_index)


def _linear_agg(x, p, w, b, relu=False):
    return pl.pallas_call(
        functools.partial(_mm_agg_body, relu=relu),
        grid=(N // BN,),
        in_specs=[_row_spec, _p0_spec, _p1_spec, _full_spec, _b_spec],
        out_specs=_row_spec,
        out_shape=jax.ShapeDtypeStruct((N, D), jnp.float32),
    )(x, p, p, w, b.reshape(1, D))


def kernel(x, edge_index, W_pre, b_pre, W1, b1, W2, b2):
    h0, packed = _linear_pre(x, W_pre, b_pre, edge_index)
    packed = packed.reshape(E)
    p = _scatter_partials(h0, packed)
    h1 = _linear_agg(h0, p, W1, b1, relu=True)
    q = _scatter_partials(h1, packed)
    return _linear_agg(h1, q, W2, b2, relu=False)


# speedup vs baseline: 1.1380x; 1.0098x over previous
import functools

import jax
import jax.numpy as jnp
from jax import lax
from jax.experimental import pallas as pl
from jax.experimental.pallas import tpu as pltpu
from jax.experimental.pallas import tpu_sc as plsc

N = 10000
E = 320000
D = 128

NC = 2
NS = 16
NW = NC * NS
EPW = E // NW
K = 128
CH = EPW // K
KT = EPW - CH * K
RPT = 624
TAIL = N - NS * RPT
ZR = 24


def _scatter_body(h_hbm, pk_hbm, out_hbm,
                  pkb, sbb, dbb, rows, pkT, sbT, dbT, rowsT, zbuf, acc,
                  semI, semG, semT):
    c = lax.axis_index("c")
    s = lax.axis_index("s")
    wid = c * NS + s
    base = wid * EPW

    def pkload(k, b):
        pltpu.async_copy(pk_hbm.at[pl.ds(base + k * K, K)], pkb[b], semI[b])

    def iwait(k, b):
        pltpu.make_async_copy(pk_hbm.at[pl.ds(base + k * K, K)], pkb[b],
                              semI[b]).wait()

    def unpack(b):
        for q in range(K // 16):
            v = pkb[b][pl.ds(q * 16, 16)]
            sbb[b][pl.ds(q * 16, 16)] = v >> 14
            dbb[b][pl.ds(q * 16, 16)] = v & 16383

    def gather(b):
        pltpu.async_copy(h_hbm.at[sbb[b]], rows[b], semG[b])

    def gwait(b):
        pltpu.make_async_copy(h_hbm.at[sbb[b]], rows[b], semG[b]).wait()

    def scat(b):
        pltpu.sync_copy(rows[b], acc.at[dbb[b]], add=True)

    pkload(0, 0)
    pkload(1, 1)
    pltpu.async_copy(pk_hbm.at[pl.ds(base + CH * K, KT)], pkT, semT)

    for r in range(ZR):
        for q in range(D // 16):
            zbuf[r, pl.ds(q * 16, 16)] = jnp.zeros((16,), jnp.float32)
    row0 = s * RPT
    for t in range(RPT // ZR):
        pltpu.async_copy(zbuf, acc.at[pl.ds(row0 + t * ZR, ZR)], semG[0])

    @pl.when(s == 0)
    def _zero_tail():
        pltpu.async_copy(zbuf.at[pl.ds(0, TAIL)], acc.at[pl.ds(NS * RPT, TAIL)],
                         semG[0])

    for t in range(RPT // ZR):
        pltpu.make_async_copy(zbuf, acc.at[pl.ds(row0 + t * ZR, ZR)],
                              semG[0]).wait()

    @pl.when(s == 0)
    def _zero_tail_wait():
        pltpu.make_async_copy(zbuf.at[pl.ds(0, TAIL)],
                              acc.at[pl.ds(NS * RPT, TAIL)], semG[0]).wait()

    iwait(0, 0)
    unpack(0)
    gather(0)
    pkload(2, 0)
    pltpu.make_async_copy(pk_hbm.at[pl.ds(base + CH * K, KT)], pkT,
                          semT).wait()
    vT = pkT[pl.ds(0, 16)]
    sbT[pl.ds(0, 16)] = vT >> 14
    dbT[pl.ds(0, 16)] = vT & 16383
    pltpu.async_copy(h_hbm.at[sbT], rowsT, semT)
    plsc.subcore_barrier()

    def pipe(j, carry):
        k0 = 2 * j
        iwait(k0 + 1, 1)
        unpack(1)
        gather(1)

        @pl.when(k0 + 3 < CH)
        def _pf1():
            pkload(k0 + 3, 1)

        gwait(0)
        scat(0)
        iwait(k0 + 2, 0)
        unpack(0)
        gather(0)

        @pl.when(k0 + 4 < CH)
        def _pf0():
            pkload(k0 + 4, 0)

        gwait(1)
        scat(1)
        return carry

    lax.fori_loop(0, CH // 2 - 1, pipe, 0)

    iwait(CH - 1, 1)
    unpack(1)
    gather(1)
    gwait(0)
    scat(0)
    gwait(1)
    scat(1)

    pltpu.make_async_copy(h_hbm.at[sbT], rowsT, semT).wait()
    pltpu.sync_copy(rowsT, acc.at[dbT], add=True)

    plsc.subcore_barrier()

    pltpu.sync_copy(acc.at[pl.ds(row0, RPT)],
                    out_hbm.at[c, pl.ds(row0, RPT)])

    @pl.when(s == 0)
    def _write_tail():
        pltpu.sync_copy(acc.at[pl.ds(NS * RPT, TAIL)],
                        out_hbm.at[c, pl.ds(NS * RPT, TAIL)])


@jax.jit
def _scatter_partials(h, packed):
    mesh = plsc.VectorSubcoreMesh(core_axis_name="c", subcore_axis_name="s")
    f = pl.kernel(
        _scatter_body,
        out_type=jax.ShapeDtypeStruct((NC, N, D), jnp.float32),
        mesh=mesh,
        scratch_types=[
            [pltpu.VMEM((K,), jnp.int32) for _ in range(2)],
            [pltpu.VMEM((K,), jnp.int32) for _ in range(2)],
            [pltpu.VMEM((K,), jnp.int32) for _ in range(2)],
            [pltpu.VMEM((K, D), jnp.float32) for _ in range(2)],
            pltpu.VMEM((KT,), jnp.int32),
            pltpu.VMEM((KT,), jnp.int32),
            pltpu.VMEM((KT,), jnp.int32),
            pltpu.VMEM((KT, D), jnp.float32),
            pltpu.VMEM((ZR, D), jnp.float32),
            pltpu.VMEM_SHARED((N, D), jnp.float32),
            [pltpu.SemaphoreType.DMA for _ in range(2)],
            [pltpu.SemaphoreType.DMA for _ in range(2)],
            pltpu.SemaphoreType.DMA,
        ],
    )
    return f(h, packed)


BN = 2000


def _mm_pre_body(x_ref, w_ref, b_ref, e_ref, o_ref, pk_ref):
    acc = lax.dot_general(x_ref[...], w_ref[...],
                          dimension_numbers=(((1,), (1,)), ((), ())),
                          preferred_element_type=jnp.float32,
                          precision=lax.Precision.HIGHEST)
    o_ref[...] = acc + b_ref[...]
    pk_ref[...] = (e_ref[0:1, :] << 14) | e_ref[1:2, :]


def _mm_agg_body(x_ref, p0_ref, p1_ref, w_ref, b_ref, o_ref, *, relu):
    hh = x_ref[...] + p0_ref[0] + p1_ref[0]
    acc = lax.dot_general(hh, w_ref[...],
                          dimension_numbers=(((1,), (1,)), ((), ())),
                          preferred_element_type=jnp.float32,
                          precision=lax.Precision.HIGHEST)
    acc = acc + b_ref[...]
    o_ref[...] = jnp.maximum(acc, 0.0) if relu else acc


_row_spec = pl.BlockSpec((BN, D), lambda i: (i, 0))
_p0_spec = pl.BlockSpec((1, BN, D), lambda i: (0, i, 0))
_p1_spec = pl.BlockSpec((1, BN, D), lambda i: (1, i, 0))
_full_spec = pl.BlockSpec((D, D), lambda i: (0, 0))
_b_spec = pl.BlockSpec((1, D), lambda i: (0, 0))


EB = E // (N // BN)


def _linear_pre(x, w, b, edge_index):
    return pl.pallas_call(
        _mm_pre_body,
        grid=(N // BN,),
        in_specs=[_row_spec, _full_spec, _b_spec,
                  pl.BlockSpec((2, EB), lambda i: (0, i))],
        out_specs=[_row_spec, pl.BlockSpec((1, EB), lambda i: (0, i))],
        out_shape=[jax.ShapeDtypeStruct((N, D), jnp.float32),
                   jax.ShapeDtypeStruct((1, E), jnp.int32)],
    )(x, w, b.reshape(1, D), edge_index)


def _linear_agg(x, p, w, b, relu=False):
    return pl.pallas_call(
        functools.partial(_mm_agg_body, relu=relu),
        grid=(N // BN,),
        in_specs=[_row_spec, _p0_spec, _p1_spec, _full_spec, _b_spec],
        out_specs=_row_spec,
        out_shape=jax.ShapeDtypeStruct((N, D), jnp.float32),
    )(x, p, p, w, b.reshape(1, D))


def kernel(x, edge_index, W_pre, b_pre, W1, b1, W2, b2):
    h0, packed = _linear_pre(x, W_pre, b_pre, edge_index)
    packed = packed.reshape(E)
    p = _scatter_partials(h0, packed)
    h1 = _linear_agg(h0, p, W1, b1, relu=True)
    q = _scatter_partials(h1, packed)
    return _linear_agg(h1, q, W2, b2, relu=False)


# speedup vs baseline: 1.1636x; 1.0225x over previous
import functools

import jax
import jax.numpy as jnp
from jax import lax
from jax.experimental import pallas as pl
from jax.experimental.pallas import tpu as pltpu
from jax.experimental.pallas import tpu_sc as plsc

N = 10000
E = 320000
D = 128

NC = 2
NS = 16
NW = NC * NS
EPW = E // NW
K = 128
CH = EPW // K
KT = EPW - CH * K
RPT = 624
TAIL = N - NS * RPT
ZR = 24


def _scatter_body(h_hbm, pk_hbm, out_hbm,
                  pkb, sbb, dbb, rows, pkT, sbT, dbT, rowsT, zbuf, acc,
                  semI, semG, semT):
    c = lax.axis_index("c")
    s = lax.axis_index("s")
    wid = c * NS + s
    base = wid * EPW

    def pkload(k, b):
        pltpu.async_copy(pk_hbm.at[pl.ds(base + k * K, K)], pkb[b], semI[b])

    def iwait(k, b):
        pltpu.make_async_copy(pk_hbm.at[pl.ds(base + k * K, K)], pkb[b],
                              semI[b]).wait()

    def unpack(b):
        for q in range(K // 16):
            v = pkb[b][pl.ds(q * 16, 16)]
            sbb[b][pl.ds(q * 16, 16)] = v >> 14
            dbb[b][pl.ds(q * 16, 16)] = v & 16383

    def gather(b):
        pltpu.async_copy(h_hbm.at[sbb[b]], rows[b], semG[b])

    def gwait(b):
        pltpu.make_async_copy(h_hbm.at[sbb[b]], rows[b], semG[b]).wait()

    def scat(b):
        pltpu.sync_copy(rows[b], acc.at[dbb[b]], add=True)

    pkload(0, 0)
    pkload(1, 1)
    pltpu.async_copy(pk_hbm.at[pl.ds(base + CH * K, KT)], pkT, semT)

    for r in range(ZR):
        for q in range(D // 16):
            zbuf[r, pl.ds(q * 16, 16)] = jnp.zeros((16,), jnp.float32)
    row0 = s * RPT
    for t in range(RPT // ZR):
        pltpu.async_copy(zbuf, acc.at[pl.ds(row0 + t * ZR, ZR)], semG[0])

    @pl.when(s == 0)
    def _zero_tail():
        pltpu.async_copy(zbuf.at[pl.ds(0, TAIL)], acc.at[pl.ds(NS * RPT, TAIL)],
                         semG[0])

    for t in range(RPT // ZR):
        pltpu.make_async_copy(zbuf, acc.at[pl.ds(row0 + t * ZR, ZR)],
                              semG[0]).wait()

    @pl.when(s == 0)
    def _zero_tail_wait():
        pltpu.make_async_copy(zbuf.at[pl.ds(0, TAIL)],
                              acc.at[pl.ds(NS * RPT, TAIL)], semG[0]).wait()

    iwait(0, 0)
    unpack(0)
    gather(0)
    pkload(2, 0)
    pltpu.make_async_copy(pk_hbm.at[pl.ds(base + CH * K, KT)], pkT,
                          semT).wait()
    vT = pkT[pl.ds(0, 16)]
    sbT[pl.ds(0, 16)] = vT >> 14
    dbT[pl.ds(0, 16)] = vT & 16383
    pltpu.async_copy(h_hbm.at[sbT], rowsT, semT)
    plsc.subcore_barrier()

    def pipe(j, carry):
        k0 = 2 * j
        iwait(k0 + 1, 1)
        unpack(1)
        gather(1)

        @pl.when(k0 + 3 < CH)
        def _pf1():
            pkload(k0 + 3, 1)

        gwait(0)
        scat(0)
        iwait(k0 + 2, 0)
        unpack(0)
        gather(0)

        @pl.when(k0 + 4 < CH)
        def _pf0():
            pkload(k0 + 4, 0)

        gwait(1)
        scat(1)
        return carry

    lax.fori_loop(0, CH // 2 - 1, pipe, 0)

    iwait(CH - 1, 1)
    unpack(1)
    gather(1)
    gwait(0)
    scat(0)
    gwait(1)
    scat(1)

    pltpu.make_async_copy(h_hbm.at[sbT], rowsT, semT).wait()
    pltpu.sync_copy(rowsT, acc.at[dbT], add=True)

    plsc.subcore_barrier()

    pltpu.sync_copy(acc.at[pl.ds(row0, RPT)],
                    out_hbm.at[c, pl.ds(row0, RPT)])

    @pl.when(s == 0)
    def _write_tail():
        pltpu.sync_copy(acc.at[pl.ds(NS * RPT, TAIL)],
                        out_hbm.at[c, pl.ds(NS * RPT, TAIL)])


@jax.jit
def _scatter_partials(h, packed):
    mesh = plsc.VectorSubcoreMesh(core_axis_name="c", subcore_axis_name="s")
    f = pl.kernel(
        _scatter_body,
        out_type=jax.ShapeDtypeStruct((NC, N, D), jnp.float32),
        mesh=mesh,
        scratch_types=[
            [pltpu.VMEM((K,), jnp.int32) for _ in range(2)],
            [pltpu.VMEM((K,), jnp.int32) for _ in range(2)],
            [pltpu.VMEM((K,), jnp.int32) for _ in range(2)],
            [pltpu.VMEM((K, D), jnp.float32) for _ in range(2)],
            pltpu.VMEM((KT,), jnp.int32),
            pltpu.VMEM((KT,), jnp.int32),
            pltpu.VMEM((KT,), jnp.int32),
            pltpu.VMEM((KT, D), jnp.float32),
            pltpu.VMEM((ZR, D), jnp.float32),
            pltpu.VMEM_SHARED((N, D), jnp.float32),
            [pltpu.SemaphoreType.DMA for _ in range(2)],
            [pltpu.SemaphoreType.DMA for _ in range(2)],
            pltpu.SemaphoreType.DMA,
        ],
    )
    return f(h, packed)


BN = 2000


def _mm_pre_body(x_ref, w_ref, b_ref, e_ref, o_ref, pk_ref):
    acc = lax.dot_general(x_ref[...], w_ref[...],
                          dimension_numbers=(((1,), (1,)), ((), ())),
                          preferred_element_type=jnp.float32,
                          precision=lax.Precision.DEFAULT)
    o_ref[...] = acc + b_ref[...]
    pk_ref[...] = (e_ref[0:1, :] << 14) | e_ref[1:2, :]


def _mm_agg_body(x_ref, p0_ref, p1_ref, w_ref, b_ref, o_ref, *, relu):
    hh = x_ref[...] + p0_ref[0] + p1_ref[0]
    acc = lax.dot_general(hh, w_ref[...],
                          dimension_numbers=(((1,), (1,)), ((), ())),
                          preferred_element_type=jnp.float32,
                          precision=lax.Precision.DEFAULT)
    acc = acc + b_ref[...]
    o_ref[...] = jnp.maximum(acc, 0.0) if relu else acc


_row_spec = pl.BlockSpec((BN, D), lambda i: (i, 0))
_p0_spec = pl.BlockSpec((1, BN, D), lambda i: (0, i, 0))
_p1_spec = pl.BlockSpec((1, BN, D), lambda i: (1, i, 0))
_full_spec = pl.BlockSpec((D, D), lambda i: (0, 0))
_b_spec = pl.BlockSpec((1, D), lambda i: (0, 0))


EB = E // (N // BN)


def _linear_pre(x, w, b, edge_index):
    return pl.pallas_call(
        _mm_pre_body,
        grid=(N // BN,),
        in_specs=[_row_spec, _full_spec, _b_spec,
                  pl.BlockSpec((2, EB), lambda i: (0, i))],
        out_specs=[_row_spec, pl.BlockSpec((1, EB), lambda i: (0, i))],
        out_shape=[jax.ShapeDtypeStruct((N, D), jnp.float32),
                   jax.ShapeDtypeStruct((1, E), jnp.int32)],
    )(x, w, b.reshape(1, D), edge_index)


def _linear_agg(x, p, w, b, relu=False):
    return pl.pallas_call(
        functools.partial(_mm_agg_body, relu=relu),
        grid=(N // BN,),
        in_specs=[_row_spec, _p0_spec, _p1_spec, _full_spec, _b_spec],
        out_specs=_row_spec,
        out_shape=jax.ShapeDtypeStruct((N, D), jnp.float32),
    )(x, p, p, w, b.reshape(1, D))


def kernel(x, edge_index, W_pre, b_pre, W1, b1, W2, b2):
    h0, packed = _linear_pre(x, W_pre, b_pre, edge_index)
    packed = packed.reshape(E)
    p = _scatter_partials(h0, packed)
    h1 = _linear_agg(h0, p, W1, b1, relu=True)
    q = _scatter_partials(h1, packed)
    return _linear_agg(h1, q, W2, b2, relu=False)
